# baseline (speedup 1.0000x reference)
import functools

import jax
import jax.numpy as jnp
from jax import lax
from jax.experimental import pallas as pl
from jax.experimental.pallas import tpu as pltpu
from jax.experimental.pallas import tpu_sc as plsc

C = 64
D = 64
EPS = 1e-20
_HIGHEST = lax.Precision.HIGHEST


def _encoder_body(x_ref, xnorm_ref, w1_ref, b1_ref, w2_ref, b2_ref, out_ref):
    xn = (x_ref[...] - xnorm_ref[0:1, :]) / xnorm_ref[1:2, :]
    h = jnp.dot(xn, w1_ref[...],
                preferred_element_type=jnp.float32) + b1_ref[...][None, :]
    h = jnp.maximum(h, 0.0)
    out_ref[...] = jnp.dot(h, w2_ref[...],
                           preferred_element_type=jnp.float32) + b2_ref[...][None, :]


def _encoder(x, XNorm, W1, b1, W2, b2):
    Bsz, x_dim = x.shape
    hidden = W1.shape[1]
    latent = W2.shape[1]
    bt = 128
    grid = (Bsz // bt,)
    return pl.pallas_call(
        _encoder_body,
        grid=grid,
        in_specs=[
            pl.BlockSpec((bt, x_dim), lambda i: (i, 0)),
            pl.BlockSpec((2, x_dim), lambda i: (0, 0)),
            pl.BlockSpec((x_dim, hidden), lambda i: (0, 0)),
            pl.BlockSpec((hidden,), lambda i: (0,)),
            pl.BlockSpec((hidden, latent), lambda i: (0, 0)),
            pl.BlockSpec((latent,), lambda i: (0,)),
        ],
        out_specs=pl.BlockSpec((bt, latent), lambda i: (i, 0)),
        out_shape=jax.ShapeDtypeStruct((Bsz, latent), jnp.float32),
        compiler_params=pltpu.CompilerParams(
            dimension_semantics=("arbitrary",)),
    )(x, XNorm, W1, b1, W2, b2)


_ROT_A = (13, 15, 26, 6)
_ROT_B = (17, 29, 16, 24)
_KS0 = 0
_KS1 = 42
_KS2 = 42 ^ 0x1BD11BDA


def _rotl(x, r):
    return lax.shift_left(x, jnp.int32(r)) | lax.shift_right_logical(
        x, jnp.int32(32 - r))


def _tf_rounds(x0, x1, rots):
    for r in rots:
        x0 = x0 + x1
        x1 = _rotl(x1, r)
        x1 = x0 ^ x1
    return x0, x1


def _tf_bits(i):
    x0 = jnp.zeros_like(i) + jnp.int32(_KS0)
    x1 = i + jnp.int32(_KS1)
    x0, x1 = _tf_rounds(x0, x1, _ROT_A)
    x0 = x0 + jnp.int32(_KS1); x1 = x1 + jnp.int32(_KS2 + 1)
    x0, x1 = _tf_rounds(x0, x1, _ROT_B)
    x0 = x0 + jnp.int32(_KS2); x1 = x1 + jnp.int32(_KS0 + 2)
    x0, x1 = _tf_rounds(x0, x1, _ROT_A)
    x0 = x0 + jnp.int32(_KS0); x1 = x1 + jnp.int32(_KS1 + 3)
    x0, x1 = _tf_rounds(x0, x1, _ROT_B)
    x0 = x0 + jnp.int32(_KS1); x1 = x1 + jnp.int32(_KS2 + 4)
    x0, x1 = _tf_rounds(x0, x1, _ROT_A)
    x0 = x0 + jnp.int32(_KS2); x1 = x1 + jnp.int32(_KS0 + 5)
    return x0 ^ x1


def _uniform_bits(i):
    bits = _tf_bits(i)
    fb = lax.shift_right_logical(bits, jnp.int32(9)) | jnp.int32(0x3F800000)
    return lax.bitcast_convert_type(fb, jnp.float32) - 1.0


def _noise_bits_sc(n, flat_offset):
    info = plsc.get_sparse_core_info()
    nw = info.num_cores * info.num_subcores
    per_w = n // nw
    blk = 16384
    n_blk = per_w // blk
    mesh = plsc.VectorSubcoreMesh(core_axis_name="c", subcore_axis_name="s")

    @functools.partial(
        pl.kernel, mesh=mesh,
        out_type=jax.ShapeDtypeStruct((n,), jnp.int32),
        scratch_types=[pltpu.VMEM((blk,), jnp.int32)],
        compiler_params=pltpu.CompilerParams(use_tc_tiling_on_sc=False),
    )
    def bits_kernel(out_hbm, buf):
        wid = lax.axis_index("s") * info.num_cores + lax.axis_index("c")
        wbase = wid * per_w
        lanes = lax.iota(jnp.int32, 16)

        def _outer(bblk, _):
            bbase = bblk * blk

            def _inner(v, _):
                i = lanes + (jnp.int32(flat_offset) + wbase + bbase + v * 16)
                buf[pl.ds(v * 16, 16)] = _tf_bits(i)
                return 0

            lax.fori_loop(0, blk // 16, _inner, 0)
            pltpu.sync_copy(buf, out_hbm.at[pl.ds(wbase + bbase, blk)])
            return 0

        lax.fori_loop(0, n_blk, _outer, 0)

    return bits_kernel()


def _argmax_from_u(u, logits, scale):
    noise = u - 0.5
    samples = scale * noise + 0.5
    g = -jnp.log(-jnp.log(samples + EPS) + EPS)
    s = logits + g

    def _half(sx, parity):
        m = jnp.max(sx, axis=-1, keepdims=True)
        i2 = lax.broadcasted_iota(jnp.int32, sx.shape, 2)
        idx = jnp.min(jnp.where(sx == m, i2, D), axis=-1)
        j = lax.broadcasted_iota(jnp.int32, idx.shape, 1)
        return (2 * j + parity) * D + idx

    oA = _half(s[:, :, :D], 0)
    oB = _half(s[:, :, D:], 1)
    return jnp.concatenate([oA, oB], axis=1)


def _sample_tc_body(knn_ref, logits_ref, offs_ref, *, bt, n_t):
    g = pl.program_id(0)
    scale = knn_ref[g // n_t]
    row_base = g * bt
    shp = (bt, C // 2, 2 * D)
    i0 = lax.broadcasted_iota(jnp.int32, shp, 0)
    i1 = lax.broadcasted_iota(jnp.int32, shp, 1)
    i2 = lax.broadcasted_iota(jnp.int32, shp, 2)
    flat = (row_base * (C * D)) + i0 * (C * D) + i1 * (2 * D) + i2
    u = _uniform_bits(flat)
    offs_ref[...] = _argmax_from_u(u, logits_ref[...], scale)


def _sample_sc_fed_body(knn_ref, logits_ref, ubits_ref, offs_in_ref, offs_ref,
                        *, g0, bt, n_t):
    del offs_in_ref
    g = pl.program_id(0) + g0
    scale = knn_ref[g // n_t]
    fb = lax.shift_right_logical(ubits_ref[...], jnp.int32(9)) \
        | jnp.int32(0x3F800000)
    u = lax.bitcast_convert_type(fb, jnp.float32) - 1.0
    offs_ref[...] = _argmax_from_u(u, logits_ref[...], scale)


def _sample(logits3, knn, ubits3, K, n_tc_rows):
    Bsz = logits3.shape[0]
    bt = 256
    n_t = Bsz // bt
    g0 = n_tc_rows // bt
    n_g = K * Bsz // bt
    offs_tc = pl.pallas_call(
        functools.partial(_sample_tc_body, bt=bt, n_t=n_t),
        grid=(g0,),
        in_specs=[
            pl.BlockSpec(memory_space=pltpu.SMEM),
            pl.BlockSpec((bt, C // 2, 2 * D), lambda g: (g % n_t, 0, 0)),
        ],
        out_specs=pl.BlockSpec((bt, C), lambda g: (g, 0)),
        out_shape=jax.ShapeDtypeStruct((K * Bsz, C), jnp.int32),
        compiler_params=pltpu.CompilerParams(
            dimension_semantics=("arbitrary",)),
    )(knn, logits3)
    offs = pl.pallas_call(
        functools.partial(_sample_sc_fed_body, g0=g0, bt=bt, n_t=n_t),
        grid=(n_g - g0,),
        in_specs=[
            pl.BlockSpec(memory_space=pltpu.SMEM),
            pl.BlockSpec((bt, C // 2, 2 * D), lambda g: ((g + g0) % n_t, 0, 0)),
            pl.BlockSpec((bt, C // 2, 2 * D), lambda g: (g, 0, 0)),
            pl.BlockSpec(memory_space=pl.ANY),
        ],
        out_specs=pl.BlockSpec((bt, C), lambda g: (g + g0, 0)),
        out_shape=jax.ShapeDtypeStruct((K * Bsz, C), jnp.int32),
        input_output_aliases={3: 0},
        compiler_params=pltpu.CompilerParams(
            dimension_semantics=("arbitrary",)),
    )(knn, logits3, ubits3, offs_tc)
    return offs


def _build_estimate_sc(offs):
    N = offs.shape[0]
    latent = C * D
    info = plsc.get_sparse_core_info()
    nw = info.num_cores * info.num_subcores
    rows_per_w = N // nw
    grp = 8
    n_grps = rows_per_w // grp
    mesh = plsc.VectorSubcoreMesh(core_axis_name="c", subcore_axis_name="s")
    offs_flat = offs.reshape(N * C)

    @functools.partial(
        pl.kernel, mesh=mesh,
        out_type=jax.ShapeDtypeStruct((N * latent,), jnp.float32),
        scratch_types=[
            pltpu.VMEM((grp * latent,), jnp.float32),
            pltpu.VMEM((grp * C,), jnp.int32),
        ],
        compiler_params=pltpu.CompilerParams(use_tc_tiling_on_sc=False,
                                             needs_layout_passes=False),
    )
    def est_kernel(offs_hbm, est_hbm, buf, offs_v):
        wid = lax.axis_index("s") * info.num_cores + lax.axis_index("c")
        base = wid * rows_per_w
        zeros16 = jnp.zeros((16,), jnp.float32)
        ones16 = jnp.ones((16,), jnp.float32)

        def _zbody(i, _):
            buf[pl.ds(i * 16, 16)] = zeros16
            return 0
        lax.fori_loop(0, grp * latent // 16, _zbody, 0)

        def _grp_body(g, _):
            rbase = base + g * grp
            pltpu.sync_copy(offs_hbm.at[pl.ds(rbase * C, grp * C)], offs_v)
            for r in range(grp):
                for j in range(C // 16):
                    ov = offs_v[pl.ds(r * C + j * 16, 16)] + (r * latent)
                    plsc.store_scatter(buf, [ov], ones16)
            pltpu.sync_copy(buf, est_hbm.at[pl.ds(rbase * latent, grp * latent)])
            for r in range(grp):
                for j in range(C // 16):
                    ov = offs_v[pl.ds(r * C + j * 16, 16)] + (r * latent)
                    plsc.store_scatter(buf, [ov], zeros16)
            return 0

        lax.fori_loop(0, n_grps, _grp_body, 0)

    return est_kernel(offs_flat).reshape(N, latent)


def _decoder_body(offs_ref, wd1_ref, bd1_ref, wd2_ref, bd2_ref, ynorm_ref,
                  y_ref):
    latent = wd1_ref.shape[0]
    offs = offs_ref[...]
    ecol = lax.broadcasted_iota(jnp.int32, (C, latent), 1) // D
    erow = lax.broadcasted_iota(jnp.int32, (C, latent), 0)
    grp = jnp.where(erow < C // 2, 2 * erow, 2 * erow - (C - 1))
    E = (ecol == grp).astype(jnp.bfloat16)
    hi = lax.shift_right_logical(offs, 8).astype(jnp.bfloat16)
    lo = (offs & 255).astype(jnp.bfloat16)
    expand = (jnp.dot(hi, E, preferred_element_type=jnp.float32) * 256.0
              + jnp.dot(lo, E, preferred_element_type=jnp.float32))
    lane = lax.broadcasted_iota(jnp.int32, expand.shape, 1)
    onehot = (expand.astype(jnp.int32) == lane).astype(jnp.bfloat16)
    hd = jnp.dot(onehot, wd1_ref[...],
                 preferred_element_type=jnp.float32) + bd1_ref[...][None, :]
    hd = jnp.maximum(hd, 0.0)
    yb = jnp.dot(hd.astype(jnp.bfloat16), wd2_ref[...],
                 preferred_element_type=jnp.float32) + bd2_ref[...][None, :]
    y_ref[...] = yb * ynorm_ref[1:2, :] + ynorm_ref[0:1, :]


def _decoder(offs, Wd1_bf, bd1, Wd2_bf, bd2, YNorm):
    N = offs.shape[0]
    latent, hidden = Wd1_bf.shape
    y_dim = Wd2_bf.shape[1]
    bt = 256
    grid = (N // bt,)
    return pl.pallas_call(
        _decoder_body,
        grid=grid,
        in_specs=[
            pl.BlockSpec((bt, C), lambda i: (i, 0)),
            pl.BlockSpec((latent, hidden), lambda i: (0, 0)),
            pl.BlockSpec((hidden,), lambda i: (0,)),
            pl.BlockSpec((hidden, y_dim), lambda i: (0, 0)),
            pl.BlockSpec((y_dim,), lambda i: (0,)),
            pl.BlockSpec((2, y_dim), lambda i: (0, 0)),
        ],
        out_specs=pl.BlockSpec((bt, y_dim), lambda i: (i, 0)),
        out_shape=jax.ShapeDtypeStruct((N, y_dim), jnp.float32),
        compiler_params=pltpu.CompilerParams(
            dimension_semantics=("arbitrary",)),
    )(offs, Wd1_bf, bd1, Wd2_bf, bd2, YNorm)


def kernel(x, knn, XNorm, YNorm, W1, b1, W2, b2, Wd1, bd1, Wd2, bd2):
    Bsz = x.shape[0]
    K = knn.shape[0]
    n_tc_rows = 9 * (K * Bsz) // 16
    n_sc = (K * Bsz - n_tc_rows) * C * D
    ubits = _noise_bits_sc(n_sc, n_tc_rows * C * D)
    ubits3 = ubits.reshape(K * Bsz - n_tc_rows, C // 2, 2 * D)

    logits = _encoder(x, XNorm, W1, b1, W2, b2)
    logits3 = logits.reshape(Bsz, C // 2, 2 * D)
    offs = _sample(logits3, knn, ubits3, K, n_tc_rows)

    y = _decoder(offs, Wd1.astype(jnp.bfloat16), bd1,
                 Wd2.astype(jnp.bfloat16), bd2, YNorm)
    estimate = _build_estimate_sc(offs)
    return (y, estimate)


# speedup vs baseline: 1.0845x; 1.0845x over previous
import functools

import jax
import jax.numpy as jnp
from jax import lax
from jax.experimental import pallas as pl
from jax.experimental.pallas import tpu as pltpu
from jax.experimental.pallas import tpu_sc as plsc

C = 64
D = 64
EPS = 1e-20
_HIGHEST = lax.Precision.HIGHEST


def _encoder_body(x_ref, xnorm_ref, w1_ref, b1_ref, w2_ref, b2_ref, out_ref):
    xn = (x_ref[...] - xnorm_ref[0:1, :]) / xnorm_ref[1:2, :]
    h = jnp.dot(xn, w1_ref[...],
                preferred_element_type=jnp.float32) + b1_ref[...][None, :]
    h = jnp.maximum(h, 0.0)
    out_ref[...] = jnp.dot(h, w2_ref[...],
                           preferred_element_type=jnp.float32) + b2_ref[...][None, :]


def _encoder(x, XNorm, W1, b1, W2, b2):
    Bsz, x_dim = x.shape
    hidden = W1.shape[1]
    latent = W2.shape[1]
    bt = 128
    grid = (Bsz // bt,)
    return pl.pallas_call(
        _encoder_body,
        grid=grid,
        in_specs=[
            pl.BlockSpec((bt, x_dim), lambda i: (i, 0)),
            pl.BlockSpec((2, x_dim), lambda i: (0, 0)),
            pl.BlockSpec((x_dim, hidden), lambda i: (0, 0)),
            pl.BlockSpec((hidden,), lambda i: (0,)),
            pl.BlockSpec((hidden, latent), lambda i: (0, 0)),
            pl.BlockSpec((latent,), lambda i: (0,)),
        ],
        out_specs=pl.BlockSpec((bt, latent), lambda i: (i, 0)),
        out_shape=jax.ShapeDtypeStruct((Bsz, latent), jnp.float32),
        compiler_params=pltpu.CompilerParams(
            dimension_semantics=("arbitrary",)),
    )(x, XNorm, W1, b1, W2, b2)


_ROT_A = (13, 15, 26, 6)
_ROT_B = (17, 29, 16, 24)
_KS0 = 0
_KS1 = 42
_KS2 = 42 ^ 0x1BD11BDA


def _rotl(x, r):
    return lax.shift_left(x, jnp.int32(r)) | lax.shift_right_logical(
        x, jnp.int32(32 - r))


def _tf_rounds(x0, x1, rots):
    for r in rots:
        x0 = x0 + x1
        x1 = _rotl(x1, r)
        x1 = x0 ^ x1
    return x0, x1


def _tf_bits(i):
    x1i = i + jnp.int32(_KS1)
    x0 = x1i
    x1 = x0 ^ _rotl(x1i, _ROT_A[0])
    x0, x1 = _tf_rounds(x0, x1, _ROT_A[1:])
    x0 = x0 + jnp.int32(_KS1); x1 = x1 + jnp.int32(_KS2 + 1)
    x0, x1 = _tf_rounds(x0, x1, _ROT_B)
    x0 = x0 + jnp.int32(_KS2); x1 = x1 + jnp.int32(_KS0 + 2)
    x0, x1 = _tf_rounds(x0, x1, _ROT_A)
    x0 = x0 + jnp.int32(_KS0); x1 = x1 + jnp.int32(_KS1 + 3)
    x0, x1 = _tf_rounds(x0, x1, _ROT_B)
    x0 = x0 + jnp.int32(_KS1); x1 = x1 + jnp.int32(_KS2 + 4)
    x0, x1 = _tf_rounds(x0, x1, _ROT_A)
    x0 = x0 + jnp.int32(_KS2); x1 = x1 + jnp.int32(_KS0 + 5)
    return x0 ^ x1


def _uniform_bits(i):
    bits = _tf_bits(i)
    fb = lax.shift_right_logical(bits, jnp.int32(9)) | jnp.int32(0x3F800000)
    return lax.bitcast_convert_type(fb, jnp.float32) - 1.0


def _noise_bits_sc(n, flat_offset):
    info = plsc.get_sparse_core_info()
    nw = info.num_cores * info.num_subcores
    per_w = n // nw
    blk = 16384
    n_blk = per_w // blk
    mesh = plsc.VectorSubcoreMesh(core_axis_name="c", subcore_axis_name="s")

    @functools.partial(
        pl.kernel, mesh=mesh,
        out_type=jax.ShapeDtypeStruct((n,), jnp.int32),
        scratch_types=[pltpu.VMEM((blk,), jnp.int32)],
        compiler_params=pltpu.CompilerParams(use_tc_tiling_on_sc=False),
    )
    def bits_kernel(out_hbm, buf):
        wid = lax.axis_index("s") * info.num_cores + lax.axis_index("c")
        wbase = wid * per_w
        lanes = lax.iota(jnp.int32, 16)

        def _outer(bblk, _):
            bbase = bblk * blk

            def _inner(v, _):
                i = lanes + (jnp.int32(flat_offset) + wbase + bbase + v * 16)
                buf[pl.ds(v * 16, 16)] = _tf_bits(i)
                return 0

            lax.fori_loop(0, blk // 16, _inner, 0)
            pltpu.sync_copy(buf, out_hbm.at[pl.ds(wbase + bbase, blk)])
            return 0

        lax.fori_loop(0, n_blk, _outer, 0)

    return bits_kernel()


def _argmax_from_u(u, logits, scale):
    noise = u - 0.5
    samples = scale * noise + 0.5
    g = -jnp.log(-jnp.log(samples + EPS) + EPS)
    s = logits + g

    def _half(sx, parity):
        m = jnp.max(sx, axis=-1, keepdims=True)
        i2 = lax.broadcasted_iota(jnp.int32, sx.shape, 2)
        idx = jnp.min(jnp.where(sx == m, i2, D), axis=-1)
        j = lax.broadcasted_iota(jnp.int32, idx.shape, 1)
        return (2 * j + parity) * D + idx

    oA = _half(s[:, :, :D], 0)
    oB = _half(s[:, :, D:], 1)
    return jnp.concatenate([oA, oB], axis=1)


def _sample_tc_body(knn_ref, logits_ref, offs_ref, *, bt, n_t):
    g = pl.program_id(0)
    scale = knn_ref[g // n_t]
    row_base = g * bt
    shp = (bt, C // 2, 2 * D)
    i0 = lax.broadcasted_iota(jnp.int32, shp, 0)
    i1 = lax.broadcasted_iota(jnp.int32, shp, 1)
    i2 = lax.broadcasted_iota(jnp.int32, shp, 2)
    flat = (row_base * (C * D)) + i0 * (C * D) + i1 * (2 * D) + i2
    u = _uniform_bits(flat)
    offs_ref[...] = _argmax_from_u(u, logits_ref[...], scale)


def _sample_sc_fed_body(knn_ref, logits_ref, ubits_ref, offs_in_ref, offs_ref,
                        *, g0, bt, n_t):
    del offs_in_ref
    g = pl.program_id(0) + g0
    scale = knn_ref[g // n_t]
    fb = lax.shift_right_logical(ubits_ref[...], jnp.int32(9)) \
        | jnp.int32(0x3F800000)
    u = lax.bitcast_convert_type(fb, jnp.float32) - 1.0
    offs_ref[...] = _argmax_from_u(u, logits_ref[...], scale)


def _sample(logits3, knn, ubits3, K, n_tc_rows):
    Bsz = logits3.shape[0]
    bt = 256
    n_t = Bsz // bt
    g0 = n_tc_rows // bt
    n_g = K * Bsz // bt
    offs_tc = pl.pallas_call(
        functools.partial(_sample_tc_body, bt=bt, n_t=n_t),
        grid=(g0,),
        in_specs=[
            pl.BlockSpec(memory_space=pltpu.SMEM),
            pl.BlockSpec((bt, C // 2, 2 * D), lambda g: (g % n_t, 0, 0)),
        ],
        out_specs=pl.BlockSpec((bt, C), lambda g: (g, 0)),
        out_shape=jax.ShapeDtypeStruct((K * Bsz, C), jnp.int32),
        compiler_params=pltpu.CompilerParams(
            dimension_semantics=("arbitrary",)),
    )(knn, logits3)
    offs = pl.pallas_call(
        functools.partial(_sample_sc_fed_body, g0=g0, bt=bt, n_t=n_t),
        grid=(n_g - g0,),
        in_specs=[
            pl.BlockSpec(memory_space=pltpu.SMEM),
            pl.BlockSpec((bt, C // 2, 2 * D), lambda g: ((g + g0) % n_t, 0, 0)),
            pl.BlockSpec((bt, C // 2, 2 * D), lambda g: (g, 0, 0)),
            pl.BlockSpec(memory_space=pl.ANY),
        ],
        out_specs=pl.BlockSpec((bt, C), lambda g: (g + g0, 0)),
        out_shape=jax.ShapeDtypeStruct((K * Bsz, C), jnp.int32),
        input_output_aliases={3: 0},
        compiler_params=pltpu.CompilerParams(
            dimension_semantics=("arbitrary",)),
    )(knn, logits3, ubits3, offs_tc)
    return offs


def _build_estimate_sc(offs):
    N = offs.shape[0]
    latent = C * D
    info = plsc.get_sparse_core_info()
    nw = info.num_cores * info.num_subcores
    rows_per_w = N // nw
    grp = 8
    n_grps = rows_per_w // grp
    mesh = plsc.VectorSubcoreMesh(core_axis_name="c", subcore_axis_name="s")
    offs_flat = offs.reshape(N * C)

    @functools.partial(
        pl.kernel, mesh=mesh,
        out_type=jax.ShapeDtypeStruct((N * latent,), jnp.float32),
        scratch_types=[
            pltpu.VMEM((grp * latent,), jnp.float32),
            pltpu.VMEM((grp * C,), jnp.int32),
        ],
        compiler_params=pltpu.CompilerParams(use_tc_tiling_on_sc=False,
                                             needs_layout_passes=False),
    )
    def est_kernel(offs_hbm, est_hbm, buf, offs_v):
        wid = lax.axis_index("s") * info.num_cores + lax.axis_index("c")
        base = wid * rows_per_w
        zeros16 = jnp.zeros((16,), jnp.float32)
        ones16 = jnp.ones((16,), jnp.float32)

        def _zbody(i, _):
            buf[pl.ds(i * 16, 16)] = zeros16
            return 0
        lax.fori_loop(0, grp * latent // 16, _zbody, 0)

        def _grp_body(g, _):
            rbase = base + g * grp
            pltpu.sync_copy(offs_hbm.at[pl.ds(rbase * C, grp * C)], offs_v)
            for r in range(grp):
                for j in range(C // 16):
                    ov = offs_v[pl.ds(r * C + j * 16, 16)] + (r * latent)
                    plsc.store_scatter(buf, [ov], ones16)
            pltpu.sync_copy(buf, est_hbm.at[pl.ds(rbase * latent, grp * latent)])
            for r in range(grp):
                for j in range(C // 16):
                    ov = offs_v[pl.ds(r * C + j * 16, 16)] + (r * latent)
                    plsc.store_scatter(buf, [ov], zeros16)
            return 0

        lax.fori_loop(0, n_grps, _grp_body, 0)

    return est_kernel(offs_flat).reshape(N, latent)


def _decoder_body(offs_ref, wd1_ref, bd1_ref, wd2_ref, bd2_ref, ynorm_ref,
                  y_ref):
    latent = wd1_ref.shape[0]
    offs = offs_ref[...]
    ecol = lax.broadcasted_iota(jnp.int32, (C, latent), 1) // D
    erow = lax.broadcasted_iota(jnp.int32, (C, latent), 0)
    grp = jnp.where(erow < C // 2, 2 * erow, 2 * erow - (C - 1))
    E = (ecol == grp).astype(jnp.bfloat16)
    hi = lax.shift_right_logical(offs, 8).astype(jnp.bfloat16)
    lo = (offs & 255).astype(jnp.bfloat16)
    expand = (jnp.dot(hi, E, preferred_element_type=jnp.float32) * 256.0
              + jnp.dot(lo, E, preferred_element_type=jnp.float32))
    lane = lax.broadcasted_iota(jnp.int32, expand.shape, 1)
    onehot = (expand.astype(jnp.int32) == lane).astype(jnp.bfloat16)
    hd = jnp.dot(onehot, wd1_ref[...],
                 preferred_element_type=jnp.float32) + bd1_ref[...][None, :]
    hd = jnp.maximum(hd, 0.0)
    yb = jnp.dot(hd.astype(jnp.bfloat16), wd2_ref[...],
                 preferred_element_type=jnp.float32) + bd2_ref[...][None, :]
    y_ref[...] = yb * ynorm_ref[1:2, :] + ynorm_ref[0:1, :]


def _decoder(offs, Wd1_bf, bd1, Wd2_bf, bd2, YNorm):
    N = offs.shape[0]
    latent, hidden = Wd1_bf.shape
    y_dim = Wd2_bf.shape[1]
    bt = 256
    grid = (N // bt,)
    return pl.pallas_call(
        _decoder_body,
        grid=grid,
        in_specs=[
            pl.BlockSpec((bt, C), lambda i: (i, 0)),
            pl.BlockSpec((latent, hidden), lambda i: (0, 0)),
            pl.BlockSpec((hidden,), lambda i: (0,)),
            pl.BlockSpec((hidden, y_dim), lambda i: (0, 0)),
            pl.BlockSpec((y_dim,), lambda i: (0,)),
            pl.BlockSpec((2, y_dim), lambda i: (0, 0)),
        ],
        out_specs=pl.BlockSpec((bt, y_dim), lambda i: (i, 0)),
        out_shape=jax.ShapeDtypeStruct((N, y_dim), jnp.float32),
        compiler_params=pltpu.CompilerParams(
            dimension_semantics=("arbitrary",)),
    )(offs, Wd1_bf, bd1, Wd2_bf, bd2, YNorm)


def kernel(x, knn, XNorm, YNorm, W1, b1, W2, b2, Wd1, bd1, Wd2, bd2):
    Bsz = x.shape[0]
    K = knn.shape[0]
    n_tc_rows = 5 * (K * Bsz) // 8
    n_sc = (K * Bsz - n_tc_rows) * C * D
    ubits = _noise_bits_sc(n_sc, n_tc_rows * C * D)
    ubits3 = ubits.reshape(K * Bsz - n_tc_rows, C // 2, 2 * D)

    logits = _encoder(x, XNorm, W1, b1, W2, b2)
    logits3 = logits.reshape(Bsz, C // 2, 2 * D)
    offs = _sample(logits3, knn, ubits3, K, n_tc_rows)

    y = _decoder(offs, Wd1.astype(jnp.bfloat16), bd1,
                 Wd2.astype(jnp.bfloat16), bd2, YNorm)
    estimate = _build_estimate_sc(offs)
    return (y, estimate)


# speedup vs baseline: 1.1199x; 1.0327x over previous
import functools

import jax
import jax.numpy as jnp
from jax import lax
from jax.experimental import pallas as pl
from jax.experimental.pallas import tpu as pltpu
from jax.experimental.pallas import tpu_sc as plsc

C = 64
D = 64
EPS = 1e-20
_HIGHEST = lax.Precision.HIGHEST


_ROT_A = (13, 15, 26, 6)
_ROT_B = (17, 29, 16, 24)
_KS0 = 0
_KS1 = 42
_KS2 = 42 ^ 0x1BD11BDA


def _rotl(x, r):
    return lax.shift_left(x, jnp.int32(r)) | lax.shift_right_logical(
        x, jnp.int32(32 - r))


def _tf_rounds(x0, x1, rots):
    for r in rots:
        x0 = x0 + x1
        x1 = _rotl(x1, r)
        x1 = x0 ^ x1
    return x0, x1


def _tf_bits(i):
    x1i = i + jnp.int32(_KS1)
    x0 = x1i
    x1 = x0 ^ _rotl(x1i, _ROT_A[0])
    x0, x1 = _tf_rounds(x0, x1, _ROT_A[1:])
    x0 = x0 + jnp.int32(_KS1); x1 = x1 + jnp.int32(_KS2 + 1)
    x0, x1 = _tf_rounds(x0, x1, _ROT_B)
    x0 = x0 + jnp.int32(_KS2); x1 = x1 + jnp.int32(_KS0 + 2)
    x0, x1 = _tf_rounds(x0, x1, _ROT_A)
    x0 = x0 + jnp.int32(_KS0); x1 = x1 + jnp.int32(_KS1 + 3)
    x0, x1 = _tf_rounds(x0, x1, _ROT_B)
    x0 = x0 + jnp.int32(_KS1); x1 = x1 + jnp.int32(_KS2 + 4)
    x0, x1 = _tf_rounds(x0, x1, _ROT_A)
    x0 = x0 + jnp.int32(_KS2); x1 = x1 + jnp.int32(_KS0 + 5)
    return x0 ^ x1


def _uniform_bits(i):
    bits = _tf_bits(i)
    fb = lax.shift_right_logical(bits, jnp.int32(9)) | jnp.int32(0x3F800000)
    return lax.bitcast_convert_type(fb, jnp.float32) - 1.0


def _noise_bits_sc(n, flat_offset):
    info = plsc.get_sparse_core_info()
    nw = info.num_cores * info.num_subcores
    per_w = n // nw
    blk = 16384
    n_blk = per_w // blk
    mesh = plsc.VectorSubcoreMesh(core_axis_name="c", subcore_axis_name="s")

    @functools.partial(
        pl.kernel, mesh=mesh,
        out_type=jax.ShapeDtypeStruct((n,), jnp.int32),
        scratch_types=[pltpu.VMEM((blk,), jnp.int32)],
        compiler_params=pltpu.CompilerParams(use_tc_tiling_on_sc=False),
    )
    def bits_kernel(out_hbm, buf):
        wid = lax.axis_index("s") * info.num_cores + lax.axis_index("c")
        wbase = wid * per_w
        lanes = lax.iota(jnp.int32, 16)

        def _outer(bblk, _):
            bbase = bblk * blk

            def _inner(v, _):
                i = lanes + (jnp.int32(flat_offset) + wbase + bbase + v * 16)
                buf[pl.ds(v * 16, 16)] = _tf_bits(i)
                return 0

            lax.fori_loop(0, blk // 16, _inner, 0)
            pltpu.sync_copy(buf, out_hbm.at[pl.ds(wbase + bbase, blk)])
            return 0

        lax.fori_loop(0, n_blk, _outer, 0)

    return bits_kernel()


def _argmax_from_u(u, logits, scale):
    noise = u - 0.5
    samples = scale * noise + 0.5
    g = -jnp.log(-jnp.log(samples + EPS) + EPS)
    s = logits + g

    def _half(sx, parity):
        m = jnp.max(sx, axis=-1, keepdims=True)
        i2 = lax.broadcasted_iota(jnp.int32, sx.shape, 2)
        idx = jnp.min(jnp.where(sx == m, i2, D), axis=-1)
        j = lax.broadcasted_iota(jnp.int32, idx.shape, 1)
        return (2 * j + parity) * D + idx

    oA = _half(s[:, :, :D], 0)
    oB = _half(s[:, :, D:], 1)
    return jnp.concatenate([oA, oB], axis=1)


def _sample_tc_body(knn_ref, x_ref, xnorm_ref, w1_ref, b1_ref, w2_ref, b2_ref,
                    offs_ref, logits_out_ref, lscratch, *, bt, n_t):
    g = pl.program_id(0)
    scale = knn_ref[g // n_t]
    row_base = g * bt

    @pl.when(g < n_t)
    def _():
        xn = (x_ref[...] - xnorm_ref[0:1, :]) / xnorm_ref[1:2, :]
        h = jnp.dot(xn.astype(jnp.bfloat16), w1_ref[...],
                    preferred_element_type=jnp.float32) + b1_ref[...][None, :]
        h = jnp.maximum(h, 0.0)
        lg = jnp.dot(h.astype(jnp.bfloat16), w2_ref[...],
                     preferred_element_type=jnp.float32) + b2_ref[...][None, :]
        lscratch[pl.ds(g * bt, bt)] = lg.reshape(bt, C // 2, 2 * D)

    lgt = lscratch[pl.ds((g % n_t) * bt, bt)]
    logits_out_ref[...] = lgt
    shp = (bt, C // 2, 2 * D)
    i0 = lax.broadcasted_iota(jnp.int32, shp, 0)
    i1 = lax.broadcasted_iota(jnp.int32, shp, 1)
    i2 = lax.broadcasted_iota(jnp.int32, shp, 2)
    flat = (row_base * (C * D)) + i0 * (C * D) + i1 * (2 * D) + i2
    u = _uniform_bits(flat)
    offs_ref[...] = _argmax_from_u(u, lgt, scale)


def _sample_sc_fed_body(knn_ref, logits_ref, ubits_ref, offs_in_ref, offs_ref,
                        *, g0, bt, n_t):
    del offs_in_ref
    g = pl.program_id(0) + g0
    scale = knn_ref[g // n_t]
    fb = lax.shift_right_logical(ubits_ref[...], jnp.int32(9)) \
        | jnp.int32(0x3F800000)
    u = lax.bitcast_convert_type(fb, jnp.float32) - 1.0
    offs_ref[...] = _argmax_from_u(u, logits_ref[...], scale)


def _sample(x, XNorm, W1_bf, b1, W2_bf, b2, knn, ubits3, K, n_tc_rows):
    Bsz, x_dim = x.shape
    hidden = W1_bf.shape[1]
    latent = W2_bf.shape[1]
    bt = 256
    n_t = Bsz // bt
    g0 = n_tc_rows // bt
    n_g = K * Bsz // bt
    offs_tc, logits3 = pl.pallas_call(
        functools.partial(_sample_tc_body, bt=bt, n_t=n_t),
        grid=(g0,),
        in_specs=[
            pl.BlockSpec(memory_space=pltpu.SMEM),
            pl.BlockSpec((bt, x_dim), lambda g: (g % n_t, 0)),
            pl.BlockSpec((2, x_dim), lambda g: (0, 0)),
            pl.BlockSpec((x_dim, hidden), lambda g: (0, 0)),
            pl.BlockSpec((hidden,), lambda g: (0,)),
            pl.BlockSpec((hidden, latent), lambda g: (0, 0)),
            pl.BlockSpec((latent,), lambda g: (0,)),
        ],
        out_specs=[
            pl.BlockSpec((bt, C), lambda g: (g, 0)),
            pl.BlockSpec((bt, C // 2, 2 * D), lambda g: (g, 0, 0)),
        ],
        out_shape=[
            jax.ShapeDtypeStruct((K * Bsz, C), jnp.int32),
            jax.ShapeDtypeStruct((g0 * bt, C // 2, 2 * D), jnp.float32),
        ],
        scratch_shapes=[pltpu.VMEM((Bsz, C // 2, 2 * D), jnp.float32)],
        compiler_params=pltpu.CompilerParams(
            dimension_semantics=("arbitrary",)),
    )(knn, x, XNorm, W1_bf, b1, W2_bf, b2)
    offs = pl.pallas_call(
        functools.partial(_sample_sc_fed_body, g0=g0, bt=bt, n_t=n_t),
        grid=(n_g - g0,),
        in_specs=[
            pl.BlockSpec(memory_space=pltpu.SMEM),
            pl.BlockSpec((bt, C // 2, 2 * D), lambda g: ((g + g0) % n_t, 0, 0)),
            pl.BlockSpec((bt, C // 2, 2 * D), lambda g: (g, 0, 0)),
            pl.BlockSpec(memory_space=pl.ANY),
        ],
        out_specs=pl.BlockSpec((bt, C), lambda g: (g + g0, 0)),
        out_shape=jax.ShapeDtypeStruct((K * Bsz, C), jnp.int32),
        input_output_aliases={3: 0},
        compiler_params=pltpu.CompilerParams(
            dimension_semantics=("arbitrary",)),
    )(knn, logits3, ubits3, offs_tc)
    return offs


def _build_estimate_sc(offs):
    N = offs.shape[0]
    latent = C * D
    info = plsc.get_sparse_core_info()
    nw = info.num_cores * info.num_subcores
    rows_per_w = N // nw
    grp = 8
    n_grps = rows_per_w // grp
    mesh = plsc.VectorSubcoreMesh(core_axis_name="c", subcore_axis_name="s")
    offs_flat = offs.reshape(N * C)

    @functools.partial(
        pl.kernel, mesh=mesh,
        out_type=jax.ShapeDtypeStruct((N * latent,), jnp.float32),
        scratch_types=[
            pltpu.VMEM((grp * latent,), jnp.float32),
            pltpu.VMEM((grp * C,), jnp.int32),
        ],
        compiler_params=pltpu.CompilerParams(use_tc_tiling_on_sc=False,
                                             needs_layout_passes=False),
    )
    def est_kernel(offs_hbm, est_hbm, buf, offs_v):
        wid = lax.axis_index("s") * info.num_cores + lax.axis_index("c")
        base = wid * rows_per_w
        zeros16 = jnp.zeros((16,), jnp.float32)
        ones16 = jnp.ones((16,), jnp.float32)

        def _zbody(i, _):
            buf[pl.ds(i * 16, 16)] = zeros16
            return 0
        lax.fori_loop(0, grp * latent // 16, _zbody, 0)

        def _grp_body(g, _):
            rbase = base + g * grp
            pltpu.sync_copy(offs_hbm.at[pl.ds(rbase * C, grp * C)], offs_v)
            for r in range(grp):
                for j in range(C // 16):
                    ov = offs_v[pl.ds(r * C + j * 16, 16)] + (r * latent)
                    plsc.store_scatter(buf, [ov], ones16)
            pltpu.sync_copy(buf, est_hbm.at[pl.ds(rbase * latent, grp * latent)])
            for r in range(grp):
                for j in range(C // 16):
                    ov = offs_v[pl.ds(r * C + j * 16, 16)] + (r * latent)
                    plsc.store_scatter(buf, [ov], zeros16)
            return 0

        lax.fori_loop(0, n_grps, _grp_body, 0)

    return est_kernel(offs_flat).reshape(N, latent)


def _decoder_body(offs_ref, wd1_ref, bd1_ref, wd2_ref, bd2_ref, ynorm_ref,
                  y_ref):
    latent = wd1_ref.shape[0]
    offs = offs_ref[...]
    ecol = lax.broadcasted_iota(jnp.int32, (C, latent), 1) // D
    erow = lax.broadcasted_iota(jnp.int32, (C, latent), 0)
    grp = jnp.where(erow < C // 2, 2 * erow, 2 * erow - (C - 1))
    E = (ecol == grp).astype(jnp.bfloat16)
    hi = lax.shift_right_logical(offs, 8).astype(jnp.bfloat16)
    lo = (offs & 255).astype(jnp.bfloat16)
    expand = (jnp.dot(hi, E, preferred_element_type=jnp.float32) * 256.0
              + jnp.dot(lo, E, preferred_element_type=jnp.float32))
    lane = lax.broadcasted_iota(jnp.int32, expand.shape, 1)
    onehot = (expand.astype(jnp.int32) == lane).astype(jnp.bfloat16)
    hd = jnp.dot(onehot, wd1_ref[...],
                 preferred_element_type=jnp.float32) + bd1_ref[...][None, :]
    hd = jnp.maximum(hd, 0.0)
    yb = jnp.dot(hd.astype(jnp.bfloat16), wd2_ref[...],
                 preferred_element_type=jnp.float32) + bd2_ref[...][None, :]
    y_ref[...] = yb * ynorm_ref[1:2, :] + ynorm_ref[0:1, :]


def _decoder(offs, Wd1_bf, bd1, Wd2_bf, bd2, YNorm):
    N = offs.shape[0]
    latent, hidden = Wd1_bf.shape
    y_dim = Wd2_bf.shape[1]
    bt = 256
    grid = (N // bt,)
    return pl.pallas_call(
        _decoder_body,
        grid=grid,
        in_specs=[
            pl.BlockSpec((bt, C), lambda i: (i, 0)),
            pl.BlockSpec((latent, hidden), lambda i: (0, 0)),
            pl.BlockSpec((hidden,), lambda i: (0,)),
            pl.BlockSpec((hidden, y_dim), lambda i: (0, 0)),
            pl.BlockSpec((y_dim,), lambda i: (0,)),
            pl.BlockSpec((2, y_dim), lambda i: (0, 0)),
        ],
        out_specs=pl.BlockSpec((bt, y_dim), lambda i: (i, 0)),
        out_shape=jax.ShapeDtypeStruct((N, y_dim), jnp.float32),
        compiler_params=pltpu.CompilerParams(
            dimension_semantics=("arbitrary",)),
    )(offs, Wd1_bf, bd1, Wd2_bf, bd2, YNorm)


def kernel(x, knn, XNorm, YNorm, W1, b1, W2, b2, Wd1, bd1, Wd2, bd2):
    Bsz = x.shape[0]
    K = knn.shape[0]
    n_tc_rows = 5 * (K * Bsz) // 8
    n_sc = (K * Bsz - n_tc_rows) * C * D
    ubits = _noise_bits_sc(n_sc, n_tc_rows * C * D)
    ubits3 = ubits.reshape(K * Bsz - n_tc_rows, C // 2, 2 * D)

    offs = _sample(x, XNorm, W1.astype(jnp.bfloat16), b1,
                   W2.astype(jnp.bfloat16), b2, knn, ubits3, K,
                   n_tc_rows)

    y = _decoder(offs, Wd1.astype(jnp.bfloat16), bd1,
                 Wd2.astype(jnp.bfloat16), bd2, YNorm)
    estimate = _build_estimate_sc(offs)
    return (y, estimate)
